# baseline (speedup 1.0000x reference)
import functools

import jax
import jax.numpy as jnp
from jax import lax
from jax.experimental import pallas as pl
from jax.experimental.pallas import tpu as pltpu
from jax.experimental.pallas import tpu_sc as plsc

N_NODES = 10000
N_EDGES = 320000
FEAT = 128
HID = 128
M_DIM = 16
N_GRAPHS = 64
TW = 136
EO_W = 24

NC = 2
NS = 16
NW = NC * NS
CHUNK = 125
NCHUNK = N_EDGES // (NW * CHUNK)

E_BLK = 2000
N_BLK = 1000


_ROWS_PER_TILE = N_NODES // NS


@functools.cache
def _sc_mesh():
    return plsc.VectorSubcoreMesh(
        core_axis_name="c", subcore_axis_name="s",
        num_cores=NC, num_subcores=NS)


@functools.cache
def _sc_gather_call():
    @functools.partial(
        pl.kernel,
        out_type=[jax.ShapeDtypeStruct((NW, NCHUNK, CHUNK, TW), jnp.float32),
                  jax.ShapeDtypeStruct((NW, NCHUNK, CHUNK, TW), jnp.float32)],
        mesh=_sc_mesh(),
        scratch_types=[pltpu.VMEM((NCHUNK, CHUNK), jnp.int32),
                       pltpu.VMEM((NCHUNK, CHUNK), jnp.int32),
                       pltpu.VMEM((CHUNK, TW), jnp.float32),
                       pltpu.VMEM((CHUNK, TW), jnp.float32),
                       pltpu.SemaphoreType.DMA,
                       pltpu.SemaphoreType.DMA],
        compiler_params=pltpu.CompilerParams(use_tc_tiling_on_sc=False),
    )
    def gather(table_hbm, src_hbm, dst_hbm, outs_hbm, outd_hbm,
               idxs_v, idxd_v, rows_s, rows_d, sem_s, sem_d):
        wid = lax.axis_index("s") * NC + lax.axis_index("c")
        pltpu.sync_copy(src_hbm.at[wid], idxs_v)
        pltpu.sync_copy(dst_hbm.at[wid], idxd_v)

        def body(j, carry):
            ga = pltpu.async_copy(table_hbm.at[idxs_v.at[j]], rows_s, sem_s)
            gb = pltpu.async_copy(table_hbm.at[idxd_v.at[j]], rows_d, sem_d)
            ga.wait()
            gb.wait()
            wa = pltpu.async_copy(rows_s, outs_hbm.at[wid, j], sem_s)
            wb = pltpu.async_copy(rows_d, outd_hbm.at[wid, j], sem_d)
            wa.wait()
            wb.wait()
            return carry

        lax.fori_loop(0, NCHUNK, body, 0)

    return gather


def _sc_gather(table, src3, dst3):
    return _sc_gather_call()(table, src3, dst3)


@functools.cache
def _sc_scatter_call():
    @functools.partial(
        pl.kernel,
        out_type=jax.ShapeDtypeStruct((NC, N_NODES, EO_W), jnp.float32),
        mesh=_sc_mesh(),
        scratch_types=[pltpu.VMEM((NCHUNK, CHUNK), jnp.int32),
                       pltpu.VMEM((CHUNK, EO_W), jnp.float32),
                       pltpu.VMEM_SHARED((N_NODES, EO_W), jnp.float32)],
        compiler_params=pltpu.CompilerParams(use_tc_tiling_on_sc=False),
    )
    def scatter(eo_hbm, dst_hbm, zeros_hbm, acc_hbm, idx_v, rows_v, acc_sp):
        c = lax.axis_index("c")
        s = lax.axis_index("s")
        wid = s * NC + c
        row0 = s * _ROWS_PER_TILE
        pltpu.sync_copy(zeros_hbm.at[pl.ds(row0, _ROWS_PER_TILE)],
                        acc_sp.at[pl.ds(row0, _ROWS_PER_TILE)])
        pltpu.sync_copy(dst_hbm.at[wid], idx_v)
        plsc.subcore_barrier()

        def body(j, carry):
            pltpu.sync_copy(eo_hbm.at[wid, j], rows_v)
            pltpu.sync_copy(rows_v, acc_sp.at[idx_v.at[j]], add=True)
            return carry

        lax.fori_loop(0, NCHUNK, body, 0)
        plsc.subcore_barrier()
        pltpu.sync_copy(acc_sp.at[pl.ds(row0, _ROWS_PER_TILE)],
                        acc_hbm.at[c, pl.ds(row0, _ROWS_PER_TILE)])

    return scatter


def _sc_scatter(eo4, dst3, zeros_acc):
    return _sc_scatter_call()(eo4, dst3, zeros_acc)


def _silu(x):
    return x * jax.nn.sigmoid(x)


def _init_body(z_ref, pos_ref, emb_ref, in_w_ref, in_b_ref, out_ref):
    oh = (z_ref[...] == lax.broadcasted_iota(jnp.int32, (N_BLK, 10), 1))
    emb_w = jnp.dot(emb_ref[...], in_w_ref[...],
                    preferred_element_type=jnp.float32)
    feats = jnp.dot(oh.astype(jnp.float32), emb_w,
                    preferred_element_type=jnp.float32) + in_b_ref[...]
    out_ref[...] = jnp.concatenate(
        [feats, pos_ref[...], jnp.zeros((N_BLK, TW - FEAT - 3), jnp.float32)],
        axis=1)


_init_call = pl.pallas_call(
    _init_body,
    grid=(N_NODES // N_BLK,),
    in_specs=[pl.BlockSpec((N_BLK, 1), lambda j: (j, 0)),
              pl.BlockSpec((N_BLK, 3), lambda j: (j, 0)),
              pl.BlockSpec((10, FEAT), lambda j: (0, 0)),
              pl.BlockSpec((FEAT, HID), lambda j: (0, 0)),
              pl.BlockSpec((1, HID), lambda j: (0, 0))],
    out_specs=pl.BlockSpec((N_BLK, TW), lambda j: (j, 0)),
    out_shape=jax.ShapeDtypeStruct((N_NODES, TW), jnp.float32),
)


def _edge_body(xs_ref, xd_ref, w1a_ref, w1b_ref, w1c_ref, b1_ref,
               w2_ref, b2_ref, cw1_ref, cb1_ref, cw2_ref, cb2_ref, out_ref):
    fi = xd_ref[:, :FEAT].astype(jnp.bfloat16)
    fj = xs_ref[:, :FEAT].astype(jnp.bfloat16)
    rel = xs_ref[:, FEAT:FEAT + 3] - xd_ref[:, FEAT:FEAT + 3]
    rel_dist = jnp.sum(rel * rel, axis=1, keepdims=True)
    pre = (jnp.dot(fi, w1a_ref[...], preferred_element_type=jnp.float32)
           + jnp.dot(fj, w1b_ref[...], preferred_element_type=jnp.float32)
           + rel_dist * w1c_ref[...] + b1_ref[...])
    h = _silu(pre).astype(jnp.bfloat16)
    m = _silu(jnp.dot(h, w2_ref[...], preferred_element_type=jnp.float32)
              + b2_ref[...])
    t = _silu(jnp.dot(m, cw1_ref[...], preferred_element_type=jnp.float32)
              + cb1_ref[...])
    cw = jnp.dot(t, cw2_ref[...], preferred_element_type=jnp.float32) + cb2_ref[...]
    out_ref[...] = jnp.concatenate(
        [m, cw * rel, jnp.zeros((E_BLK, EO_W - M_DIM - 3), jnp.float32)],
        axis=1)


_EDGE_IN = HID * 2 + 1

_edge_call = pl.pallas_call(
    _edge_body,
    grid=(N_EDGES // E_BLK,),
    in_specs=[pl.BlockSpec((E_BLK, TW), lambda j: (j, 0)),
              pl.BlockSpec((E_BLK, TW), lambda j: (j, 0)),
              pl.BlockSpec((HID, _EDGE_IN * 2), lambda j: (0, 0)),
              pl.BlockSpec((HID, _EDGE_IN * 2), lambda j: (0, 0)),
              pl.BlockSpec((1, _EDGE_IN * 2), lambda j: (0, 0)),
              pl.BlockSpec((1, _EDGE_IN * 2), lambda j: (0, 0)),
              pl.BlockSpec((_EDGE_IN * 2, M_DIM), lambda j: (0, 0)),
              pl.BlockSpec((1, M_DIM), lambda j: (0, 0)),
              pl.BlockSpec((M_DIM, M_DIM * 4), lambda j: (0, 0)),
              pl.BlockSpec((1, M_DIM * 4), lambda j: (0, 0)),
              pl.BlockSpec((M_DIM * 4, 1), lambda j: (0, 0)),
              pl.BlockSpec((1, 1), lambda j: (0, 0))],
    out_specs=pl.BlockSpec((E_BLK, EO_W), lambda j: (j, 0)),
    out_shape=jax.ShapeDtypeStruct((N_EDGES, EO_W), jnp.float32),
)


def _node_body(t_ref, a0_ref, a1_ref, nw1_ref, nb1_ref, nw2_ref, nb2_ref,
               out_ref):
    feats = t_ref[:, :FEAT]
    coors = t_ref[:, FEAT:FEAT + 3]
    acc = a0_ref[...] + a1_ref[...]
    m_i = acc[:, :M_DIM]
    mhat = acc[:, M_DIM:M_DIM + 3]
    nh = _silu(jnp.dot(jnp.concatenate([feats, m_i], axis=1), nw1_ref[...],
                       preferred_element_type=jnp.float32) + nb1_ref[...])
    feats_out = feats + jnp.dot(nh, nw2_ref[...],
                                preferred_element_type=jnp.float32) + nb2_ref[...]
    out_ref[...] = jnp.concatenate(
        [feats_out, coors + mhat,
         jnp.zeros((N_BLK, TW - FEAT - 3), jnp.float32)], axis=1)


_node_call = pl.pallas_call(
    _node_body,
    grid=(N_NODES // N_BLK,),
    in_specs=[pl.BlockSpec((N_BLK, TW), lambda j: (j, 0)),
              pl.BlockSpec((N_BLK, EO_W), lambda j: (j, 0)),
              pl.BlockSpec((N_BLK, EO_W), lambda j: (j, 0)),
              pl.BlockSpec((HID + M_DIM, HID * 2), lambda j: (0, 0)),
              pl.BlockSpec((1, HID * 2), lambda j: (0, 0)),
              pl.BlockSpec((HID * 2, HID), lambda j: (0, 0)),
              pl.BlockSpec((1, HID), lambda j: (0, 0))],
    out_specs=pl.BlockSpec((N_BLK, TW), lambda j: (j, 0)),
    out_shape=jax.ShapeDtypeStruct((N_NODES, TW), jnp.float32),
)


def _pool_body(t_ref, b_ref, hw1_ref, hb1_ref, hw2_ref, hb2_ref, res_ref,
               gv_scr):
    j = pl.program_id(0)
    oh = (b_ref[...] == lax.broadcasted_iota(jnp.int32, (N_BLK, N_GRAPHS), 1))
    part = lax.dot_general(oh.astype(jnp.float32), t_ref[:, :FEAT],
                           (((0,), (0,)), ((), ())),
                           preferred_element_type=jnp.float32)

    @pl.when(j == 0)
    def _():
        gv_scr[...] = jnp.zeros((N_GRAPHS, FEAT), jnp.float32)

    gv_scr[...] += part

    @pl.when(j == N_NODES // N_BLK - 1)
    def _():
        hh = _silu(jnp.dot(gv_scr[...], hw1_ref[...],
                           preferred_element_type=jnp.float32) + hb1_ref[...])
        res_ref[...] = (jnp.dot(hh, hw2_ref[...],
                                preferred_element_type=jnp.float32)
                        + hb2_ref[...])


_pool_call = pl.pallas_call(
    _pool_body,
    grid=(N_NODES // N_BLK,),
    in_specs=[pl.BlockSpec((N_BLK, TW), lambda j: (j, 0)),
              pl.BlockSpec((N_BLK, 1), lambda j: (j, 0)),
              pl.BlockSpec((HID, HID), lambda j: (0, 0)),
              pl.BlockSpec((1, HID), lambda j: (0, 0)),
              pl.BlockSpec((HID, 1), lambda j: (0, 0)),
              pl.BlockSpec((1, 1), lambda j: (0, 0))],
    out_specs=pl.BlockSpec((N_GRAPHS, 1), lambda j: (0, 0)),
    out_shape=jax.ShapeDtypeStruct((N_GRAPHS, 1), jnp.float32),
    scratch_shapes=[pltpu.VMEM((N_GRAPHS, FEAT), jnp.float32)],
)


def kernel(z, pos, batch, edge_index, params):
    f32 = jnp.float32
    src3 = edge_index[0].astype(jnp.int32).reshape(NW, NCHUNK, CHUNK)
    dst3 = edge_index[1].astype(jnp.int32).reshape(NW, NCHUNK, CHUNK)
    zeros_acc = jnp.zeros((N_NODES, EO_W), f32)
    z_f = z.astype(jnp.int32).reshape(N_NODES, 1)
    batch_f = batch.astype(jnp.int32).reshape(N_NODES, 1)
    p = params

    table = _init_call(z_f, pos, p['emb'], p['in_w'],
                       p['in_b'].reshape(1, HID))

    for lp in p['layers']:
        xs4, xd4 = _sc_gather(table, src3, dst3)
        eo = _edge_call(xs4.reshape(N_EDGES, TW), xd4.reshape(N_EDGES, TW),
                        lp['edge_w1'][:HID].astype(jnp.bfloat16),
                        lp['edge_w1'][HID:2 * HID].astype(jnp.bfloat16),
                        lp['edge_w1'][2 * HID:].reshape(1, _EDGE_IN * 2),
                        lp['edge_b1'].reshape(1, _EDGE_IN * 2),
                        lp['edge_w2'].astype(jnp.bfloat16),
                        lp['edge_b2'].reshape(1, M_DIM),
                        lp['coors_w1'],
                        lp['coors_b1'].reshape(1, M_DIM * 4),
                        lp['coors_w2'],
                        lp['coors_b2'].reshape(1, 1))
        acc = _sc_scatter(eo.reshape(NW, NCHUNK, CHUNK, EO_W), dst3, zeros_acc)
        table = _node_call(table, acc[0], acc[1],
                           lp['node_w1'], lp['node_b1'].reshape(1, HID * 2),
                           lp['node_w2'], lp['node_b2'].reshape(1, HID))

    return _pool_call(table, batch_f, p['head_w1'],
                      p['head_b1'].reshape(1, HID),
                      p['head_w2'], p['head_b2'].reshape(1, 1))


# speedup vs baseline: 1.0032x; 1.0032x over previous
import functools

import jax
import jax.numpy as jnp
from jax import lax
from jax.experimental import pallas as pl
from jax.experimental.pallas import tpu as pltpu
from jax.experimental.pallas import tpu_sc as plsc

N_NODES = 10000
N_EDGES = 320000
FEAT = 128
HID = 128
M_DIM = 16
N_GRAPHS = 64
TW = 136
EO_W = 24

NC = 2
NS = 16
NW = NC * NS
CHUNK = 125
NCHUNK = N_EDGES // (NW * CHUNK)

E_BLK = 2000
N_BLK = 1000


_ROWS_PER_TILE = N_NODES // NS


@functools.cache
def _sc_mesh():
    return plsc.VectorSubcoreMesh(
        core_axis_name="c", subcore_axis_name="s",
        num_cores=NC, num_subcores=NS)


@functools.cache
def _sc_gather_call():
    @functools.partial(
        pl.kernel,
        out_type=[jax.ShapeDtypeStruct((N_EDGES, TW), jnp.float32),
                  jax.ShapeDtypeStruct((N_EDGES, TW), jnp.float32)],
        mesh=_sc_mesh(),
        scratch_types=[pltpu.VMEM((NCHUNK, CHUNK), jnp.int32),
                       pltpu.VMEM((NCHUNK, CHUNK), jnp.int32),
                       pltpu.VMEM((CHUNK, TW), jnp.float32),
                       pltpu.VMEM((CHUNK, TW), jnp.float32),
                       pltpu.SemaphoreType.DMA,
                       pltpu.SemaphoreType.DMA],
        compiler_params=pltpu.CompilerParams(use_tc_tiling_on_sc=False),
    )
    def gather(table_hbm, src_hbm, dst_hbm, outs_hbm, outd_hbm,
               idxs_v, idxd_v, rows_s, rows_d, sem_s, sem_d):
        wid = lax.axis_index("s") * NC + lax.axis_index("c")
        base = wid * (NCHUNK * CHUNK)
        pltpu.sync_copy(src_hbm.at[wid], idxs_v)
        pltpu.sync_copy(dst_hbm.at[wid], idxd_v)

        def body(j, carry):
            row = base + j * CHUNK
            ga = pltpu.async_copy(table_hbm.at[idxs_v.at[j]], rows_s, sem_s)
            gb = pltpu.async_copy(table_hbm.at[idxd_v.at[j]], rows_d, sem_d)
            ga.wait()
            gb.wait()
            wa = pltpu.async_copy(rows_s, outs_hbm.at[pl.ds(row, CHUNK)], sem_s)
            wb = pltpu.async_copy(rows_d, outd_hbm.at[pl.ds(row, CHUNK)], sem_d)
            wa.wait()
            wb.wait()
            return carry

        lax.fori_loop(0, NCHUNK, body, 0)

    return gather


def _sc_gather(table, src3, dst3):
    return _sc_gather_call()(table, src3, dst3)


@functools.cache
def _sc_scatter_call():
    @functools.partial(
        pl.kernel,
        out_type=jax.ShapeDtypeStruct((NC * N_NODES, EO_W), jnp.float32),
        mesh=_sc_mesh(),
        scratch_types=[pltpu.VMEM((NCHUNK, CHUNK), jnp.int32),
                       pltpu.VMEM((CHUNK, EO_W), jnp.float32),
                       pltpu.VMEM_SHARED((N_NODES, EO_W), jnp.float32)],
        compiler_params=pltpu.CompilerParams(use_tc_tiling_on_sc=False),
    )
    def scatter(eo_hbm, dst_hbm, zeros_hbm, acc_hbm, idx_v, rows_v, acc_sp):
        c = lax.axis_index("c")
        s = lax.axis_index("s")
        wid = s * NC + c
        base = wid * (NCHUNK * CHUNK)
        row0 = s * _ROWS_PER_TILE
        pltpu.sync_copy(zeros_hbm.at[pl.ds(row0, _ROWS_PER_TILE)],
                        acc_sp.at[pl.ds(row0, _ROWS_PER_TILE)])
        pltpu.sync_copy(dst_hbm.at[wid], idx_v)
        plsc.subcore_barrier()

        def body(j, carry):
            pltpu.sync_copy(eo_hbm.at[pl.ds(base + j * CHUNK, CHUNK)], rows_v)
            pltpu.sync_copy(rows_v, acc_sp.at[idx_v.at[j]], add=True)
            return carry

        lax.fori_loop(0, NCHUNK, body, 0)
        plsc.subcore_barrier()
        pltpu.sync_copy(acc_sp.at[pl.ds(row0, _ROWS_PER_TILE)],
                        acc_hbm.at[pl.ds(c * N_NODES + row0, _ROWS_PER_TILE)])

    return scatter


def _sc_scatter(eo4, dst3, zeros_acc):
    return _sc_scatter_call()(eo4, dst3, zeros_acc)


def _silu(x):
    return x * jax.nn.sigmoid(x)


def _init_body(z_ref, pos_ref, emb_ref, in_w_ref, in_b_ref, out_ref):
    oh = (z_ref[...] == lax.broadcasted_iota(jnp.int32, (N_BLK, 10), 1))
    emb_w = jnp.dot(emb_ref[...], in_w_ref[...],
                    preferred_element_type=jnp.float32)
    feats = jnp.dot(oh.astype(jnp.float32), emb_w,
                    preferred_element_type=jnp.float32) + in_b_ref[...]
    out_ref[...] = jnp.concatenate(
        [feats, pos_ref[...], jnp.zeros((N_BLK, TW - FEAT - 3), jnp.float32)],
        axis=1)


_init_call = pl.pallas_call(
    _init_body,
    grid=(N_NODES // N_BLK,),
    in_specs=[pl.BlockSpec((N_BLK, 1), lambda j: (j, 0)),
              pl.BlockSpec((N_BLK, 3), lambda j: (j, 0)),
              pl.BlockSpec((10, FEAT), lambda j: (0, 0)),
              pl.BlockSpec((FEAT, HID), lambda j: (0, 0)),
              pl.BlockSpec((1, HID), lambda j: (0, 0))],
    out_specs=pl.BlockSpec((N_BLK, TW), lambda j: (j, 0)),
    out_shape=jax.ShapeDtypeStruct((N_NODES, TW), jnp.float32),
)


def _edge_body(xs_ref, xd_ref, w1a_ref, w1b_ref, w1c_ref, b1_ref,
               w2_ref, b2_ref, cw1_ref, cb1_ref, cw2_ref, cb2_ref, out_ref):
    fi = xd_ref[:, :FEAT].astype(jnp.bfloat16)
    fj = xs_ref[:, :FEAT].astype(jnp.bfloat16)
    rel = xs_ref[:, FEAT:FEAT + 3] - xd_ref[:, FEAT:FEAT + 3]
    rel_dist = jnp.sum(rel * rel, axis=1, keepdims=True)
    pre = (jnp.dot(fi, w1a_ref[...], preferred_element_type=jnp.float32)
           + jnp.dot(fj, w1b_ref[...], preferred_element_type=jnp.float32)
           + rel_dist * w1c_ref[...] + b1_ref[...])
    h = _silu(pre).astype(jnp.bfloat16)
    m = _silu(jnp.dot(h, w2_ref[...], preferred_element_type=jnp.float32)
              + b2_ref[...])
    t = _silu(jnp.dot(m, cw1_ref[...], preferred_element_type=jnp.float32)
              + cb1_ref[...])
    cw = jnp.dot(t, cw2_ref[...], preferred_element_type=jnp.float32) + cb2_ref[...]
    out_ref[...] = jnp.concatenate(
        [m, cw * rel, jnp.zeros((E_BLK, EO_W - M_DIM - 3), jnp.float32)],
        axis=1)


_EDGE_IN = HID * 2 + 1

_edge_call = pl.pallas_call(
    _edge_body,
    grid=(N_EDGES // E_BLK,),
    in_specs=[pl.BlockSpec((E_BLK, TW), lambda j: (j, 0)),
              pl.BlockSpec((E_BLK, TW), lambda j: (j, 0)),
              pl.BlockSpec((HID, _EDGE_IN * 2), lambda j: (0, 0)),
              pl.BlockSpec((HID, _EDGE_IN * 2), lambda j: (0, 0)),
              pl.BlockSpec((1, _EDGE_IN * 2), lambda j: (0, 0)),
              pl.BlockSpec((1, _EDGE_IN * 2), lambda j: (0, 0)),
              pl.BlockSpec((_EDGE_IN * 2, M_DIM), lambda j: (0, 0)),
              pl.BlockSpec((1, M_DIM), lambda j: (0, 0)),
              pl.BlockSpec((M_DIM, M_DIM * 4), lambda j: (0, 0)),
              pl.BlockSpec((1, M_DIM * 4), lambda j: (0, 0)),
              pl.BlockSpec((M_DIM * 4, 1), lambda j: (0, 0)),
              pl.BlockSpec((1, 1), lambda j: (0, 0))],
    out_specs=pl.BlockSpec((E_BLK, EO_W), lambda j: (j, 0)),
    out_shape=jax.ShapeDtypeStruct((N_EDGES, EO_W), jnp.float32),
)


def _node_body(t_ref, a0_ref, a1_ref, nw1_ref, nb1_ref, nw2_ref, nb2_ref,
               out_ref):
    feats = t_ref[:, :FEAT]
    coors = t_ref[:, FEAT:FEAT + 3]
    acc = a0_ref[...] + a1_ref[...]
    m_i = acc[:, :M_DIM]
    mhat = acc[:, M_DIM:M_DIM + 3]
    nh = _silu(jnp.dot(jnp.concatenate([feats, m_i], axis=1), nw1_ref[...],
                       preferred_element_type=jnp.float32) + nb1_ref[...])
    feats_out = feats + jnp.dot(nh, nw2_ref[...],
                                preferred_element_type=jnp.float32) + nb2_ref[...]
    out_ref[...] = jnp.concatenate(
        [feats_out, coors + mhat,
         jnp.zeros((N_BLK, TW - FEAT - 3), jnp.float32)], axis=1)


_node_call = pl.pallas_call(
    _node_body,
    grid=(N_NODES // N_BLK,),
    in_specs=[pl.BlockSpec((N_BLK, TW), lambda j: (j, 0)),
              pl.BlockSpec((N_BLK, EO_W), lambda j: (j, 0)),
              pl.BlockSpec((N_BLK, EO_W),
                           lambda j: (j + N_NODES // N_BLK, 0)),
              pl.BlockSpec((HID + M_DIM, HID * 2), lambda j: (0, 0)),
              pl.BlockSpec((1, HID * 2), lambda j: (0, 0)),
              pl.BlockSpec((HID * 2, HID), lambda j: (0, 0)),
              pl.BlockSpec((1, HID), lambda j: (0, 0))],
    out_specs=pl.BlockSpec((N_BLK, TW), lambda j: (j, 0)),
    out_shape=jax.ShapeDtypeStruct((N_NODES, TW), jnp.float32),
)


def _pool_body(t_ref, b_ref, hw1_ref, hb1_ref, hw2_ref, hb2_ref, res_ref,
               gv_scr):
    j = pl.program_id(0)
    oh = (b_ref[...] == lax.broadcasted_iota(jnp.int32, (N_BLK, N_GRAPHS), 1))
    part = lax.dot_general(oh.astype(jnp.float32), t_ref[:, :FEAT],
                           (((0,), (0,)), ((), ())),
                           preferred_element_type=jnp.float32)

    @pl.when(j == 0)
    def _():
        gv_scr[...] = jnp.zeros((N_GRAPHS, FEAT), jnp.float32)

    gv_scr[...] += part

    @pl.when(j == N_NODES // N_BLK - 1)
    def _():
        hh = _silu(jnp.dot(gv_scr[...], hw1_ref[...],
                           preferred_element_type=jnp.float32) + hb1_ref[...])
        res_ref[...] = (jnp.dot(hh, hw2_ref[...],
                                preferred_element_type=jnp.float32)
                        + hb2_ref[...])


_pool_call = pl.pallas_call(
    _pool_body,
    grid=(N_NODES // N_BLK,),
    in_specs=[pl.BlockSpec((N_BLK, TW), lambda j: (j, 0)),
              pl.BlockSpec((N_BLK, 1), lambda j: (j, 0)),
              pl.BlockSpec((HID, HID), lambda j: (0, 0)),
              pl.BlockSpec((1, HID), lambda j: (0, 0)),
              pl.BlockSpec((HID, 1), lambda j: (0, 0)),
              pl.BlockSpec((1, 1), lambda j: (0, 0))],
    out_specs=pl.BlockSpec((N_GRAPHS, 1), lambda j: (0, 0)),
    out_shape=jax.ShapeDtypeStruct((N_GRAPHS, 1), jnp.float32),
    scratch_shapes=[pltpu.VMEM((N_GRAPHS, FEAT), jnp.float32)],
)


def kernel(z, pos, batch, edge_index, params):
    f32 = jnp.float32
    src3 = edge_index[0].astype(jnp.int32).reshape(NW, NCHUNK, CHUNK)
    dst3 = edge_index[1].astype(jnp.int32).reshape(NW, NCHUNK, CHUNK)
    zeros_acc = jnp.zeros((N_NODES, EO_W), f32)
    z_f = z.astype(jnp.int32).reshape(N_NODES, 1)
    batch_f = batch.astype(jnp.int32).reshape(N_NODES, 1)
    p = params

    table = _init_call(z_f, pos, p['emb'], p['in_w'],
                       p['in_b'].reshape(1, HID))

    for lp in p['layers']:
        xs, xd = _sc_gather(table, src3, dst3)
        eo = _edge_call(xs, xd,
                        lp['edge_w1'][:HID].astype(jnp.bfloat16),
                        lp['edge_w1'][HID:2 * HID].astype(jnp.bfloat16),
                        lp['edge_w1'][2 * HID:].reshape(1, _EDGE_IN * 2),
                        lp['edge_b1'].reshape(1, _EDGE_IN * 2),
                        lp['edge_w2'].astype(jnp.bfloat16),
                        lp['edge_b2'].reshape(1, M_DIM),
                        lp['coors_w1'],
                        lp['coors_b1'].reshape(1, M_DIM * 4),
                        lp['coors_w2'],
                        lp['coors_b2'].reshape(1, 1))
        acc = _sc_scatter(eo, dst3, zeros_acc)
        table = _node_call(table, acc, acc,
                           lp['node_w1'], lp['node_b1'].reshape(1, HID * 2),
                           lp['node_w2'], lp['node_b2'].reshape(1, HID))

    return _pool_call(table, batch_f, p['head_w1'],
                      p['head_b1'].reshape(1, HID),
                      p['head_w2'], p['head_b2'].reshape(1, 1))


# speedup vs baseline: 1.4716x; 1.4670x over previous
import functools

import jax
import jax.numpy as jnp
from jax import lax
from jax.experimental import pallas as pl
from jax.experimental.pallas import tpu as pltpu
from jax.experimental.pallas import tpu_sc as plsc

N_NODES = 10000
N_EDGES = 320000
FEAT = 128
HID = 128
M_DIM = 16
N_GRAPHS = 64
CW = 8
EO_W = 24

NC = 2
NS = 16
NW = NC * NS
E_PER_W = N_EDGES // NW
CHUNK = 80
NCHUNK = E_PER_W // CHUNK

E_BLK = 2000
N_BLK = 1000

_ROWS_PER_TILE = N_NODES // NS


@functools.cache
def _sc_mesh():
    return plsc.VectorSubcoreMesh(
        core_axis_name="c", subcore_axis_name="s",
        num_cores=NC, num_subcores=NS)


@functools.cache
def _sc_gather_feats_call():
    @functools.partial(
        pl.kernel,
        out_type=[jax.ShapeDtypeStruct((N_EDGES, FEAT), jnp.float32),
                  jax.ShapeDtypeStruct((N_EDGES, FEAT), jnp.float32)],
        mesh=_sc_mesh(),
        scratch_types=[pltpu.VMEM((E_PER_W,), jnp.int32),
                       pltpu.VMEM((E_PER_W,), jnp.int32),
                       pltpu.VMEM((CHUNK, FEAT), jnp.float32),
                       pltpu.VMEM((CHUNK, FEAT), jnp.float32),
                       pltpu.SemaphoreType.DMA,
                       pltpu.SemaphoreType.DMA],
    )
    def gather(table_hbm, src_hbm, dst_hbm, outs_hbm, outd_hbm,
               idxs_v, idxd_v, rows_s, rows_d, sem_s, sem_d):
        wid = lax.axis_index("s") * NC + lax.axis_index("c")
        base = wid * E_PER_W
        pltpu.sync_copy(src_hbm.at[pl.ds(base, E_PER_W)], idxs_v)
        pltpu.sync_copy(dst_hbm.at[pl.ds(base, E_PER_W)], idxd_v)

        def body(j, carry):
            off = j * CHUNK
            ga = pltpu.async_copy(
                table_hbm.at[idxs_v.at[pl.ds(off, CHUNK)]], rows_s, sem_s)
            gb = pltpu.async_copy(
                table_hbm.at[idxd_v.at[pl.ds(off, CHUNK)]], rows_d, sem_d)
            ga.wait()
            gb.wait()
            wa = pltpu.async_copy(rows_s, outs_hbm.at[pl.ds(base + off, CHUNK)],
                                  sem_s)
            wb = pltpu.async_copy(rows_d, outd_hbm.at[pl.ds(base + off, CHUNK)],
                                  sem_d)
            wa.wait()
            wb.wait()
            return carry

        lax.fori_loop(0, NCHUNK, body, 0)

    return gather


@functools.cache
def _sc_gather_coors_call():
    @functools.partial(
        pl.kernel,
        out_type=[jax.ShapeDtypeStruct((N_EDGES, CW), jnp.float32),
                  jax.ShapeDtypeStruct((N_EDGES, CW), jnp.float32)],
        mesh=_sc_mesh(),
        scratch_types=[pltpu.VMEM((E_PER_W,), jnp.int32),
                       pltpu.VMEM((E_PER_W,), jnp.int32),
                       pltpu.VMEM((CHUNK, CW), jnp.float32),
                       pltpu.VMEM((CHUNK, CW), jnp.float32),
                       pltpu.SemaphoreType.DMA,
                       pltpu.SemaphoreType.DMA],
        compiler_params=pltpu.CompilerParams(use_tc_tiling_on_sc=False),
    )
    def gather(table_hbm, src_hbm, dst_hbm, outs_hbm, outd_hbm,
               idxs_v, idxd_v, rows_s, rows_d, sem_s, sem_d):
        wid = lax.axis_index("s") * NC + lax.axis_index("c")
        base = wid * E_PER_W
        pltpu.sync_copy(src_hbm.at[pl.ds(base, E_PER_W)], idxs_v)
        pltpu.sync_copy(dst_hbm.at[pl.ds(base, E_PER_W)], idxd_v)

        def body(j, carry):
            off = j * CHUNK
            ga = pltpu.async_copy(
                table_hbm.at[idxs_v.at[pl.ds(off, CHUNK)]], rows_s, sem_s)
            gb = pltpu.async_copy(
                table_hbm.at[idxd_v.at[pl.ds(off, CHUNK)]], rows_d, sem_d)
            ga.wait()
            gb.wait()
            wa = pltpu.async_copy(rows_s, outs_hbm.at[pl.ds(base + off, CHUNK)],
                                  sem_s)
            wb = pltpu.async_copy(rows_d, outd_hbm.at[pl.ds(base + off, CHUNK)],
                                  sem_d)
            wa.wait()
            wb.wait()
            return carry

        lax.fori_loop(0, NCHUNK, body, 0)

    return gather


@functools.cache
def _sc_scatter_call():
    @functools.partial(
        pl.kernel,
        out_type=jax.ShapeDtypeStruct((NC * N_NODES, EO_W), jnp.float32),
        mesh=_sc_mesh(),
        scratch_types=[pltpu.VMEM((NCHUNK, CHUNK), jnp.int32),
                       pltpu.VMEM((CHUNK, EO_W), jnp.float32),
                       pltpu.VMEM_SHARED((N_NODES, EO_W), jnp.float32)],
        compiler_params=pltpu.CompilerParams(use_tc_tiling_on_sc=False),
    )
    def scatter(eo_hbm, dst_hbm, zeros_hbm, acc_hbm, idx_v, rows_v, acc_sp):
        c = lax.axis_index("c")
        s = lax.axis_index("s")
        wid = s * NC + c
        base = wid * E_PER_W
        row0 = s * _ROWS_PER_TILE
        pltpu.sync_copy(zeros_hbm.at[pl.ds(row0, _ROWS_PER_TILE)],
                        acc_sp.at[pl.ds(row0, _ROWS_PER_TILE)])
        pltpu.sync_copy(dst_hbm.at[wid], idx_v)
        plsc.subcore_barrier()

        def body(j, carry):
            pltpu.sync_copy(eo_hbm.at[pl.ds(base + j * CHUNK, CHUNK)], rows_v)
            pltpu.sync_copy(rows_v, acc_sp.at[idx_v.at[j]], add=True)
            return carry

        lax.fori_loop(0, NCHUNK, body, 0)
        plsc.subcore_barrier()
        pltpu.sync_copy(acc_sp.at[pl.ds(row0, _ROWS_PER_TILE)],
                        acc_hbm.at[pl.ds(c * N_NODES + row0, _ROWS_PER_TILE)])

    return scatter


def _silu(x):
    return x * jax.nn.sigmoid(x)


def _init_body(z_ref, pos_ref, emb_ref, in_w_ref, in_b_ref, f_ref, c_ref):
    oh = (z_ref[...] == lax.broadcasted_iota(jnp.int32, (N_BLK, 10), 1))
    emb_w = jnp.dot(emb_ref[...], in_w_ref[...],
                    preferred_element_type=jnp.float32)
    f_ref[...] = jnp.dot(oh.astype(jnp.float32), emb_w,
                         preferred_element_type=jnp.float32) + in_b_ref[...]
    c_ref[...] = jnp.concatenate(
        [pos_ref[...], jnp.zeros((N_BLK, CW - 3), jnp.float32)], axis=1)


_init_call = pl.pallas_call(
    _init_body,
    grid=(N_NODES // N_BLK,),
    in_specs=[pl.BlockSpec((N_BLK, 1), lambda j: (j, 0)),
              pl.BlockSpec((N_BLK, 3), lambda j: (j, 0)),
              pl.BlockSpec((10, FEAT), lambda j: (0, 0)),
              pl.BlockSpec((FEAT, HID), lambda j: (0, 0)),
              pl.BlockSpec((1, HID), lambda j: (0, 0))],
    out_specs=[pl.BlockSpec((N_BLK, HID), lambda j: (j, 0)),
               pl.BlockSpec((N_BLK, CW), lambda j: (j, 0))],
    out_shape=[jax.ShapeDtypeStruct((N_NODES, HID), jnp.float32),
               jax.ShapeDtypeStruct((N_NODES, CW), jnp.float32)],
)


def _edge_body(xsf_ref, xdf_ref, xsc_ref, xdc_ref,
               w1a_ref, w1b_ref, w1c_ref, b1_ref,
               w2_ref, b2_ref, cw1_ref, cb1_ref, cw2_ref, cb2_ref, out_ref):
    fi = xdf_ref[...].astype(jnp.bfloat16)
    fj = xsf_ref[...].astype(jnp.bfloat16)
    rel = xsc_ref[:, :3] - xdc_ref[:, :3]
    rel_dist = jnp.sum(rel * rel, axis=1, keepdims=True)
    pre = (jnp.dot(fi, w1a_ref[...], preferred_element_type=jnp.float32)
           + jnp.dot(fj, w1b_ref[...], preferred_element_type=jnp.float32)
           + rel_dist * w1c_ref[...] + b1_ref[...])
    h = _silu(pre).astype(jnp.bfloat16)
    m = _silu(jnp.dot(h, w2_ref[...], preferred_element_type=jnp.float32)
              + b2_ref[...])
    t = _silu(jnp.dot(m, cw1_ref[...], preferred_element_type=jnp.float32)
              + cb1_ref[...])
    cw = jnp.dot(t, cw2_ref[...], preferred_element_type=jnp.float32) + cb2_ref[...]
    out_ref[...] = jnp.concatenate(
        [m, cw * rel, jnp.zeros((E_BLK, EO_W - M_DIM - 3), jnp.float32)],
        axis=1)


_EDGE_IN = HID * 2 + 1

_edge_call = pl.pallas_call(
    _edge_body,
    grid=(N_EDGES // E_BLK,),
    in_specs=[pl.BlockSpec((E_BLK, FEAT), lambda j: (j, 0)),
              pl.BlockSpec((E_BLK, FEAT), lambda j: (j, 0)),
              pl.BlockSpec((E_BLK, CW), lambda j: (j, 0)),
              pl.BlockSpec((E_BLK, CW), lambda j: (j, 0)),
              pl.BlockSpec((HID, _EDGE_IN * 2), lambda j: (0, 0)),
              pl.BlockSpec((HID, _EDGE_IN * 2), lambda j: (0, 0)),
              pl.BlockSpec((1, _EDGE_IN * 2), lambda j: (0, 0)),
              pl.BlockSpec((1, _EDGE_IN * 2), lambda j: (0, 0)),
              pl.BlockSpec((_EDGE_IN * 2, M_DIM), lambda j: (0, 0)),
              pl.BlockSpec((1, M_DIM), lambda j: (0, 0)),
              pl.BlockSpec((M_DIM, M_DIM * 4), lambda j: (0, 0)),
              pl.BlockSpec((1, M_DIM * 4), lambda j: (0, 0)),
              pl.BlockSpec((M_DIM * 4, 1), lambda j: (0, 0)),
              pl.BlockSpec((1, 1), lambda j: (0, 0))],
    out_specs=pl.BlockSpec((E_BLK, EO_W), lambda j: (j, 0)),
    out_shape=jax.ShapeDtypeStruct((N_EDGES, EO_W), jnp.float32),
)


def _node_body(f_ref, c_ref, a0_ref, a1_ref, nw1_ref, nb1_ref, nw2_ref,
               nb2_ref, fo_ref, co_ref):
    feats = f_ref[...]
    acc = a0_ref[...] + a1_ref[...]
    m_i = acc[:, :M_DIM]
    mhat = acc[:, M_DIM:M_DIM + 3]
    nh = _silu(jnp.dot(jnp.concatenate([feats, m_i], axis=1), nw1_ref[...],
                       preferred_element_type=jnp.float32) + nb1_ref[...])
    fo_ref[...] = feats + jnp.dot(nh, nw2_ref[...],
                                  preferred_element_type=jnp.float32) + nb2_ref[...]
    co_ref[...] = jnp.concatenate(
        [c_ref[:, :3] + mhat, jnp.zeros((N_BLK, CW - 3), jnp.float32)], axis=1)


_node_call = pl.pallas_call(
    _node_body,
    grid=(N_NODES // N_BLK,),
    in_specs=[pl.BlockSpec((N_BLK, HID), lambda j: (j, 0)),
              pl.BlockSpec((N_BLK, CW), lambda j: (j, 0)),
              pl.BlockSpec((N_BLK, EO_W), lambda j: (j, 0)),
              pl.BlockSpec((N_BLK, EO_W),
                           lambda j: (j + N_NODES // N_BLK, 0)),
              pl.BlockSpec((HID + M_DIM, HID * 2), lambda j: (0, 0)),
              pl.BlockSpec((1, HID * 2), lambda j: (0, 0)),
              pl.BlockSpec((HID * 2, HID), lambda j: (0, 0)),
              pl.BlockSpec((1, HID), lambda j: (0, 0))],
    out_specs=[pl.BlockSpec((N_BLK, HID), lambda j: (j, 0)),
               pl.BlockSpec((N_BLK, CW), lambda j: (j, 0))],
    out_shape=[jax.ShapeDtypeStruct((N_NODES, HID), jnp.float32),
               jax.ShapeDtypeStruct((N_NODES, CW), jnp.float32)],
)


def _pool_body(f_ref, b_ref, hw1_ref, hb1_ref, hw2_ref, hb2_ref, res_ref,
               gv_scr):
    j = pl.program_id(0)
    oh = (b_ref[...] == lax.broadcasted_iota(jnp.int32, (N_BLK, N_GRAPHS), 1))
    part = lax.dot_general(oh.astype(jnp.float32), f_ref[...],
                           (((0,), (0,)), ((), ())),
                           preferred_element_type=jnp.float32)

    @pl.when(j == 0)
    def _():
        gv_scr[...] = jnp.zeros((N_GRAPHS, FEAT), jnp.float32)

    gv_scr[...] += part

    @pl.when(j == N_NODES // N_BLK - 1)
    def _():
        hh = _silu(jnp.dot(gv_scr[...], hw1_ref[...],
                           preferred_element_type=jnp.float32) + hb1_ref[...])
        res_ref[...] = (jnp.dot(hh, hw2_ref[...],
                                preferred_element_type=jnp.float32)
                        + hb2_ref[...])


_pool_call = pl.pallas_call(
    _pool_body,
    grid=(N_NODES // N_BLK,),
    in_specs=[pl.BlockSpec((N_BLK, FEAT), lambda j: (j, 0)),
              pl.BlockSpec((N_BLK, 1), lambda j: (j, 0)),
              pl.BlockSpec((HID, HID), lambda j: (0, 0)),
              pl.BlockSpec((1, HID), lambda j: (0, 0)),
              pl.BlockSpec((HID, 1), lambda j: (0, 0)),
              pl.BlockSpec((1, 1), lambda j: (0, 0))],
    out_specs=pl.BlockSpec((N_GRAPHS, 1), lambda j: (0, 0)),
    out_shape=jax.ShapeDtypeStruct((N_GRAPHS, 1), jnp.float32),
    scratch_shapes=[pltpu.VMEM((N_GRAPHS, FEAT), jnp.float32)],
)


def _sc_gather_feats(table, src1, dst1):
    return _sc_gather_feats_call()(table, src1, dst1)


def _sc_gather_coors(table, src1, dst1):
    return _sc_gather_coors_call()(table, src1, dst1)


def _sc_scatter(eo, dst3, zeros_acc):
    return _sc_scatter_call()(eo, dst3, zeros_acc)


def kernel(z, pos, batch, edge_index, params):
    f32 = jnp.float32
    src1 = edge_index[0].astype(jnp.int32)
    dst1 = edge_index[1].astype(jnp.int32)
    dst3 = dst1.reshape(NW, NCHUNK, CHUNK)
    zeros_acc = jnp.zeros((N_NODES, EO_W), f32)
    z_i = z.astype(jnp.int32).reshape(N_NODES, 1)
    batch_i = batch.astype(jnp.int32).reshape(N_NODES, 1)
    p = params

    feats, coors = _init_call(z_i, pos, p['emb'], p['in_w'],
                              p['in_b'].reshape(1, HID))

    for lp in p['layers']:
        xsf, xdf = _sc_gather_feats(feats, src1, dst1)
        xsc, xdc = _sc_gather_coors(coors, src1, dst1)
        eo = _edge_call(xsf, xdf, xsc, xdc,
                        lp['edge_w1'][:HID].astype(jnp.bfloat16),
                        lp['edge_w1'][HID:2 * HID].astype(jnp.bfloat16),
                        lp['edge_w1'][2 * HID:].reshape(1, _EDGE_IN * 2),
                        lp['edge_b1'].reshape(1, _EDGE_IN * 2),
                        lp['edge_w2'].astype(jnp.bfloat16),
                        lp['edge_b2'].reshape(1, M_DIM),
                        lp['coors_w1'],
                        lp['coors_b1'].reshape(1, M_DIM * 4),
                        lp['coors_w2'],
                        lp['coors_b2'].reshape(1, 1))
        acc = _sc_scatter(eo, dst3, zeros_acc)
        feats, coors = _node_call(feats, coors, acc, acc,
                                  lp['node_w1'], lp['node_b1'].reshape(1, HID * 2),
                                  lp['node_w2'], lp['node_b2'].reshape(1, HID))

    return _pool_call(feats, batch_i, p['head_w1'],
                      p['head_b1'].reshape(1, HID),
                      p['head_w2'], p['head_b2'].reshape(1, 1))


# speedup vs baseline: 1.4840x; 1.0084x over previous
import functools

import jax
import jax.numpy as jnp
from jax import lax
from jax.experimental import pallas as pl
from jax.experimental.pallas import tpu as pltpu
from jax.experimental.pallas import tpu_sc as plsc

N_NODES = 10000
N_EDGES = 320000
FEAT = 128
HID = 128
M_DIM = 16
N_GRAPHS = 64
CW = 8
EO_W = 24

NC = 2
NS = 16
NW = NC * NS
E_PER_W = N_EDGES // NW
CHUNK = 80
NCHUNK = E_PER_W // CHUNK

E_BLK = 2000
N_BLK = 1000

_ROWS_PER_TILE = N_NODES // NS


@functools.cache
def _sc_mesh():
    return plsc.VectorSubcoreMesh(
        core_axis_name="c", subcore_axis_name="s",
        num_cores=NC, num_subcores=NS)


@functools.cache
def _sc_gather_call():
    @functools.partial(
        pl.kernel,
        out_type=[jax.ShapeDtypeStruct((N_EDGES, FEAT), jnp.float32),
                  jax.ShapeDtypeStruct((N_EDGES, FEAT), jnp.float32),
                  jax.ShapeDtypeStruct((N_EDGES, FEAT), jnp.float32)],
        mesh=_sc_mesh(),
        scratch_types=[pltpu.VMEM((E_PER_W,), jnp.int32),
                       pltpu.VMEM((E_PER_W,), jnp.int32),
                       pltpu.VMEM((CHUNK, FEAT), jnp.float32),
                       pltpu.VMEM((CHUNK, FEAT), jnp.float32),
                       pltpu.VMEM((CHUNK, CW), jnp.float32),
                       pltpu.VMEM((CHUNK, CW), jnp.float32),
                       pltpu.SemaphoreType.DMA,
                       pltpu.SemaphoreType.DMA,
                       pltpu.SemaphoreType.DMA,
                       pltpu.SemaphoreType.DMA],
        compiler_params=pltpu.CompilerParams(use_tc_tiling_on_sc=False),
    )
    def gather(feats_hbm, coors_hbm, src_hbm, dst_hbm,
               outs_hbm, outd_hbm, outc_hbm,
               idxs_v, idxd_v, rows_fs, rows_fd, rows_cs, rows_cd,
               sem_fs, sem_fd, sem_cs, sem_cd):
        wid = lax.axis_index("s") * NC + lax.axis_index("c")
        base = wid * E_PER_W
        pltpu.sync_copy(src_hbm.at[pl.ds(base, E_PER_W)], idxs_v)
        pltpu.sync_copy(dst_hbm.at[pl.ds(base, E_PER_W)], idxd_v)

        def body(j, carry):
            off = j * CHUNK
            row = base + off
            g1 = pltpu.async_copy(
                feats_hbm.at[idxs_v.at[pl.ds(off, CHUNK)]], rows_fs, sem_fs)
            g2 = pltpu.async_copy(
                feats_hbm.at[idxd_v.at[pl.ds(off, CHUNK)]], rows_fd, sem_fd)
            g3 = pltpu.async_copy(
                coors_hbm.at[idxs_v.at[pl.ds(off, CHUNK)]], rows_cs, sem_cs)
            g4 = pltpu.async_copy(
                coors_hbm.at[idxd_v.at[pl.ds(off, CHUNK)]], rows_cd, sem_cd)
            g1.wait()
            g2.wait()
            g3.wait()
            g4.wait()
            w1 = pltpu.async_copy(rows_fs, outs_hbm.at[pl.ds(row, CHUNK)],
                                  sem_fs)
            w2 = pltpu.async_copy(rows_fd, outd_hbm.at[pl.ds(row, CHUNK)],
                                  sem_fd)
            w3 = pltpu.async_copy(
                rows_cs, outc_hbm.at[pl.ds(row, CHUNK), pl.ds(0, CW)], sem_cs)
            w4 = pltpu.async_copy(
                rows_cd, outc_hbm.at[pl.ds(row, CHUNK), pl.ds(CW, CW)], sem_cd)
            w1.wait()
            w2.wait()
            w3.wait()
            w4.wait()
            return carry

        lax.fori_loop(0, NCHUNK, body, 0)

    return gather


@functools.cache
def _sc_scatter_call():
    @functools.partial(
        pl.kernel,
        out_type=jax.ShapeDtypeStruct((NC * N_NODES, EO_W), jnp.float32),
        mesh=_sc_mesh(),
        scratch_types=[pltpu.VMEM((NCHUNK, CHUNK), jnp.int32),
                       pltpu.VMEM((CHUNK, EO_W), jnp.float32),
                       pltpu.VMEM_SHARED((N_NODES, EO_W), jnp.float32)],
        compiler_params=pltpu.CompilerParams(use_tc_tiling_on_sc=False),
    )
    def scatter(eo_hbm, dst_hbm, zeros_hbm, acc_hbm, idx_v, rows_v, acc_sp):
        c = lax.axis_index("c")
        s = lax.axis_index("s")
        wid = s * NC + c
        base = wid * E_PER_W
        row0 = s * _ROWS_PER_TILE
        pltpu.sync_copy(zeros_hbm.at[pl.ds(row0, _ROWS_PER_TILE)],
                        acc_sp.at[pl.ds(row0, _ROWS_PER_TILE)])
        pltpu.sync_copy(dst_hbm.at[wid], idx_v)
        plsc.subcore_barrier()

        def body(j, carry):
            pltpu.sync_copy(
                eo_hbm.at[pl.ds(base + j * CHUNK, CHUNK), pl.ds(0, EO_W)],
                rows_v)
            pltpu.sync_copy(rows_v, acc_sp.at[idx_v.at[j]], add=True)
            return carry

        lax.fori_loop(0, NCHUNK, body, 0)
        plsc.subcore_barrier()
        pltpu.sync_copy(acc_sp.at[pl.ds(row0, _ROWS_PER_TILE)],
                        acc_hbm.at[pl.ds(c * N_NODES + row0, _ROWS_PER_TILE)])

    return scatter


def _silu(x):
    return x * jax.nn.sigmoid(x)


def _init_body(z_ref, pos_ref, emb_ref, in_w_ref, in_b_ref, f_ref, c_ref):
    oh = (z_ref[...] == lax.broadcasted_iota(jnp.int32, (N_BLK, 10), 1))
    emb_w = jnp.dot(emb_ref[...], in_w_ref[...],
                    preferred_element_type=jnp.float32)
    f_ref[...] = jnp.dot(oh.astype(jnp.float32), emb_w,
                         preferred_element_type=jnp.float32) + in_b_ref[...]
    c_ref[...] = jnp.concatenate(
        [pos_ref[...], jnp.zeros((N_BLK, CW - 3), jnp.float32)], axis=1)


_init_call = pl.pallas_call(
    _init_body,
    grid=(N_NODES // N_BLK,),
    in_specs=[pl.BlockSpec((N_BLK, 1), lambda j: (j, 0)),
              pl.BlockSpec((N_BLK, 3), lambda j: (j, 0)),
              pl.BlockSpec((10, FEAT), lambda j: (0, 0)),
              pl.BlockSpec((FEAT, HID), lambda j: (0, 0)),
              pl.BlockSpec((1, HID), lambda j: (0, 0))],
    out_specs=[pl.BlockSpec((N_BLK, HID), lambda j: (j, 0)),
               pl.BlockSpec((N_BLK, CW), lambda j: (j, 0))],
    out_shape=[jax.ShapeDtypeStruct((N_NODES, HID), jnp.float32),
               jax.ShapeDtypeStruct((N_NODES, CW), jnp.float32)],
)


def _edge_body(xsf_ref, xdf_ref, xcc_ref,
               w1a_ref, w1b_ref, w1c_ref, b1_ref,
               w2_ref, b2_ref, cw1_ref, cb1_ref, cw2_ref, cb2_ref, out_ref):
    fi = xdf_ref[...].astype(jnp.bfloat16)
    fj = xsf_ref[...].astype(jnp.bfloat16)
    rel = xcc_ref[:, :3] - xcc_ref[:, CW:CW + 3]
    rel_dist = jnp.sum(rel * rel, axis=1, keepdims=True)
    pre = (jnp.dot(fi, w1a_ref[...], preferred_element_type=jnp.float32)
           + jnp.dot(fj, w1b_ref[...], preferred_element_type=jnp.float32)
           + rel_dist * w1c_ref[...] + b1_ref[...])
    h = _silu(pre).astype(jnp.bfloat16)
    m = _silu(jnp.dot(h, w2_ref[...], preferred_element_type=jnp.float32)
              + b2_ref[...])
    t = _silu(jnp.dot(m, cw1_ref[...], preferred_element_type=jnp.float32)
              + cb1_ref[...])
    cw = jnp.dot(t, cw2_ref[...], preferred_element_type=jnp.float32) + cb2_ref[...]
    out_ref[...] = jnp.concatenate(
        [m, cw * rel, jnp.zeros((E_BLK, FEAT - M_DIM - 3), jnp.float32)],
        axis=1)


_EDGE_IN = HID * 2 + 1

_edge_call = pl.pallas_call(
    _edge_body,
    grid=(N_EDGES // E_BLK,),
    in_specs=[pl.BlockSpec((E_BLK, FEAT), lambda j: (j, 0)),
              pl.BlockSpec((E_BLK, FEAT), lambda j: (j, 0)),
              pl.BlockSpec((E_BLK, FEAT), lambda j: (j, 0)),
              pl.BlockSpec((HID, _EDGE_IN * 2), lambda j: (0, 0)),
              pl.BlockSpec((HID, _EDGE_IN * 2), lambda j: (0, 0)),
              pl.BlockSpec((1, _EDGE_IN * 2), lambda j: (0, 0)),
              pl.BlockSpec((1, _EDGE_IN * 2), lambda j: (0, 0)),
              pl.BlockSpec((_EDGE_IN * 2, M_DIM), lambda j: (0, 0)),
              pl.BlockSpec((1, M_DIM), lambda j: (0, 0)),
              pl.BlockSpec((M_DIM, M_DIM * 4), lambda j: (0, 0)),
              pl.BlockSpec((1, M_DIM * 4), lambda j: (0, 0)),
              pl.BlockSpec((M_DIM * 4, 1), lambda j: (0, 0)),
              pl.BlockSpec((1, 1), lambda j: (0, 0))],
    out_specs=pl.BlockSpec((E_BLK, FEAT), lambda j: (j, 0)),
    out_shape=jax.ShapeDtypeStruct((N_EDGES, FEAT), jnp.float32),
)


def _node_body(f_ref, c_ref, a0_ref, a1_ref, nw1_ref, nb1_ref, nw2_ref,
               nb2_ref, fo_ref, co_ref):
    feats = f_ref[...]
    acc = a0_ref[...] + a1_ref[...]
    m_i = acc[:, :M_DIM]
    mhat = acc[:, M_DIM:M_DIM + 3]
    nh = _silu(jnp.dot(jnp.concatenate([feats, m_i], axis=1), nw1_ref[...],
                       preferred_element_type=jnp.float32) + nb1_ref[...])
    fo_ref[...] = feats + jnp.dot(nh, nw2_ref[...],
                                  preferred_element_type=jnp.float32) + nb2_ref[...]
    co_ref[...] = jnp.concatenate(
        [c_ref[:, :3] + mhat, jnp.zeros((N_BLK, CW - 3), jnp.float32)], axis=1)


_node_call = pl.pallas_call(
    _node_body,
    grid=(N_NODES // N_BLK,),
    in_specs=[pl.BlockSpec((N_BLK, HID), lambda j: (j, 0)),
              pl.BlockSpec((N_BLK, CW), lambda j: (j, 0)),
              pl.BlockSpec((N_BLK, EO_W), lambda j: (j, 0)),
              pl.BlockSpec((N_BLK, EO_W),
                           lambda j: (j + N_NODES // N_BLK, 0)),
              pl.BlockSpec((HID + M_DIM, HID * 2), lambda j: (0, 0)),
              pl.BlockSpec((1, HID * 2), lambda j: (0, 0)),
              pl.BlockSpec((HID * 2, HID), lambda j: (0, 0)),
              pl.BlockSpec((1, HID), lambda j: (0, 0))],
    out_specs=[pl.BlockSpec((N_BLK, HID), lambda j: (j, 0)),
               pl.BlockSpec((N_BLK, CW), lambda j: (j, 0))],
    out_shape=[jax.ShapeDtypeStruct((N_NODES, HID), jnp.float32),
               jax.ShapeDtypeStruct((N_NODES, CW), jnp.float32)],
)


def _pool_body(f_ref, b_ref, hw1_ref, hb1_ref, hw2_ref, hb2_ref, res_ref,
               gv_scr):
    j = pl.program_id(0)
    oh = (b_ref[...] == lax.broadcasted_iota(jnp.int32, (N_BLK, N_GRAPHS), 1))
    part = lax.dot_general(oh.astype(jnp.float32), f_ref[...],
                           (((0,), (0,)), ((), ())),
                           preferred_element_type=jnp.float32)

    @pl.when(j == 0)
    def _():
        gv_scr[...] = jnp.zeros((N_GRAPHS, FEAT), jnp.float32)

    gv_scr[...] += part

    @pl.when(j == N_NODES // N_BLK - 1)
    def _():
        hh = _silu(jnp.dot(gv_scr[...], hw1_ref[...],
                           preferred_element_type=jnp.float32) + hb1_ref[...])
        res_ref[...] = (jnp.dot(hh, hw2_ref[...],
                                preferred_element_type=jnp.float32)
                        + hb2_ref[...])


_pool_call = pl.pallas_call(
    _pool_body,
    grid=(N_NODES // N_BLK,),
    in_specs=[pl.BlockSpec((N_BLK, FEAT), lambda j: (j, 0)),
              pl.BlockSpec((N_BLK, 1), lambda j: (j, 0)),
              pl.BlockSpec((HID, HID), lambda j: (0, 0)),
              pl.BlockSpec((1, HID), lambda j: (0, 0)),
              pl.BlockSpec((HID, 1), lambda j: (0, 0)),
              pl.BlockSpec((1, 1), lambda j: (0, 0))],
    out_specs=pl.BlockSpec((N_GRAPHS, 1), lambda j: (0, 0)),
    out_shape=jax.ShapeDtypeStruct((N_GRAPHS, 1), jnp.float32),
    scratch_shapes=[pltpu.VMEM((N_GRAPHS, FEAT), jnp.float32)],
)


def _sc_gather(feats, coors, src1, dst1):
    return _sc_gather_call()(feats, coors, src1, dst1)


def _sc_scatter(eo, dst3, zeros_acc):
    return _sc_scatter_call()(eo, dst3, zeros_acc)


def kernel(z, pos, batch, edge_index, params):
    f32 = jnp.float32
    src1 = edge_index[0].astype(jnp.int32)
    dst1 = edge_index[1].astype(jnp.int32)
    dst3 = dst1.reshape(NW, NCHUNK, CHUNK)
    zeros_acc = jnp.zeros((N_NODES, EO_W), f32)
    z_i = z.astype(jnp.int32).reshape(N_NODES, 1)
    batch_i = batch.astype(jnp.int32).reshape(N_NODES, 1)
    p = params

    feats, coors = _init_call(z_i, pos, p['emb'], p['in_w'],
                              p['in_b'].reshape(1, HID))

    for lp in p['layers']:
        xsf, xdf, xcc = _sc_gather(feats, coors, src1, dst1)
        eo = _edge_call(xsf, xdf, xcc,
                        lp['edge_w1'][:HID].astype(jnp.bfloat16),
                        lp['edge_w1'][HID:2 * HID].astype(jnp.bfloat16),
                        lp['edge_w1'][2 * HID:].reshape(1, _EDGE_IN * 2),
                        lp['edge_b1'].reshape(1, _EDGE_IN * 2),
                        lp['edge_w2'].astype(jnp.bfloat16),
                        lp['edge_b2'].reshape(1, M_DIM),
                        lp['coors_w1'],
                        lp['coors_b1'].reshape(1, M_DIM * 4),
                        lp['coors_w2'],
                        lp['coors_b2'].reshape(1, 1))
        acc = _sc_scatter(eo, dst3, zeros_acc)
        feats, coors = _node_call(feats, coors, acc, acc,
                                  lp['node_w1'], lp['node_b1'].reshape(1, HID * 2),
                                  lp['node_w2'], lp['node_b2'].reshape(1, HID))

    return _pool_call(feats, batch_i, p['head_w1'],
                      p['head_b1'].reshape(1, HID),
                      p['head_w2'], p['head_b2'].reshape(1, 1))


# speedup vs baseline: 1.9045x; 1.2834x over previous
import functools

import jax
import jax.numpy as jnp
from jax import lax
from jax.experimental import pallas as pl
from jax.experimental.pallas import tpu as pltpu
from jax.experimental.pallas import tpu_sc as plsc

N_NODES = 10000
N_EDGES = 320000
FEAT = 128
HID = 128
M_DIM = 16
N_GRAPHS = 64
CW = 8
EO_W = 24

NC = 2
NS = 16
NW = NC * NS
E_PER_W = N_EDGES // NW
CHUNK = 80
NCHUNK = E_PER_W // CHUNK

E_BLK = 2000
N_BLK = 1000

_ROWS_PER_TILE = N_NODES // NS


@functools.cache
def _sc_mesh():
    return plsc.VectorSubcoreMesh(
        core_axis_name="c", subcore_axis_name="s",
        num_cores=NC, num_subcores=NS)


@functools.cache
def _sc_gather_call():
    @functools.partial(
        pl.kernel,
        out_type=[jax.ShapeDtypeStruct((N_EDGES, FEAT), jnp.float32),
                  jax.ShapeDtypeStruct((N_EDGES, FEAT), jnp.float32),
                  jax.ShapeDtypeStruct((N_EDGES, FEAT), jnp.float32)],
        mesh=_sc_mesh(),
        scratch_types=[pltpu.VMEM((E_PER_W,), jnp.int32),
                       pltpu.VMEM((E_PER_W,), jnp.int32),
                       pltpu.VMEM((CHUNK, FEAT), jnp.float32),
                       pltpu.VMEM((CHUNK, FEAT), jnp.float32),
                       pltpu.VMEM((CHUNK, CW), jnp.float32),
                       pltpu.VMEM((CHUNK, CW), jnp.float32),
                       pltpu.SemaphoreType.DMA,
                       pltpu.SemaphoreType.DMA,
                       pltpu.SemaphoreType.DMA,
                       pltpu.SemaphoreType.DMA],
        compiler_params=pltpu.CompilerParams(use_tc_tiling_on_sc=False),
    )
    def gather(feats_hbm, coors_hbm, src_hbm, dst_hbm,
               outs_hbm, outd_hbm, outc_hbm,
               idxs_v, idxd_v, rows_fs, rows_fd, rows_cs, rows_cd,
               sem_fs, sem_fd, sem_cs, sem_cd):
        wid = lax.axis_index("s") * NC + lax.axis_index("c")
        base = wid * E_PER_W
        pltpu.sync_copy(src_hbm.at[pl.ds(base, E_PER_W)], idxs_v)
        pltpu.sync_copy(dst_hbm.at[pl.ds(base, E_PER_W)], idxd_v)

        def body(j, carry):
            off = j * CHUNK
            row = base + off
            g1 = pltpu.async_copy(
                feats_hbm.at[idxs_v.at[pl.ds(off, CHUNK)]], rows_fs, sem_fs)
            g2 = pltpu.async_copy(
                feats_hbm.at[idxd_v.at[pl.ds(off, CHUNK)]], rows_fd, sem_fd)
            g3 = pltpu.async_copy(
                coors_hbm.at[idxs_v.at[pl.ds(off, CHUNK)]], rows_cs, sem_cs)
            g4 = pltpu.async_copy(
                coors_hbm.at[idxd_v.at[pl.ds(off, CHUNK)]], rows_cd, sem_cd)
            g1.wait()
            g2.wait()
            g3.wait()
            g4.wait()
            w1 = pltpu.async_copy(rows_fs, outs_hbm.at[pl.ds(row, CHUNK)],
                                  sem_fs)
            w2 = pltpu.async_copy(rows_fd, outd_hbm.at[pl.ds(row, CHUNK)],
                                  sem_fd)
            w3 = pltpu.async_copy(
                rows_cs, outc_hbm.at[pl.ds(row, CHUNK), pl.ds(0, CW)], sem_cs)
            w4 = pltpu.async_copy(
                rows_cd, outc_hbm.at[pl.ds(row, CHUNK), pl.ds(CW, CW)], sem_cd)
            w1.wait()
            w2.wait()
            w3.wait()
            w4.wait()
            return carry

        lax.fori_loop(0, NCHUNK, body, 0)

    return gather


@functools.cache
def _sc_scatter_call():
    @functools.partial(
        pl.kernel,
        out_type=jax.ShapeDtypeStruct((NC * N_NODES, EO_W), jnp.float32),
        mesh=_sc_mesh(),
        scratch_types=[pltpu.VMEM((NCHUNK, CHUNK), jnp.int32),
                       pltpu.VMEM((CHUNK, EO_W), jnp.float32),
                       pltpu.VMEM_SHARED((N_NODES, EO_W), jnp.float32)],
        compiler_params=pltpu.CompilerParams(use_tc_tiling_on_sc=False),
    )
    def scatter(eo_hbm, dst_hbm, zeros_hbm, acc_hbm, idx_v, rows_v, acc_sp):
        c = lax.axis_index("c")
        s = lax.axis_index("s")
        wid = s * NC + c
        base = wid * E_PER_W
        row0 = s * _ROWS_PER_TILE
        pltpu.sync_copy(zeros_hbm.at[pl.ds(row0, _ROWS_PER_TILE)],
                        acc_sp.at[pl.ds(row0, _ROWS_PER_TILE)])
        pltpu.sync_copy(dst_hbm.at[wid], idx_v)
        plsc.subcore_barrier()

        def body(j, carry):
            pltpu.sync_copy(
                eo_hbm.at[pl.ds(base + j * CHUNK, CHUNK), pl.ds(0, EO_W)],
                rows_v)
            pltpu.sync_copy(rows_v, acc_sp.at[idx_v.at[j]], add=True)
            return carry

        lax.fori_loop(0, NCHUNK, body, 0)
        plsc.subcore_barrier()
        pltpu.sync_copy(acc_sp.at[pl.ds(row0, _ROWS_PER_TILE)],
                        acc_hbm.at[pl.ds(c * N_NODES + row0, _ROWS_PER_TILE)])

    return scatter


def _silu(x):
    return x * jax.nn.sigmoid(x)


def _init_body(z_ref, pos_ref, emb_ref, in_w_ref, in_b_ref, f_ref, c_ref):
    z = z_ref[...]
    emb_sel = jnp.zeros((N_BLK, FEAT), jnp.float32)
    for k in range(10):
        sel = (z == k).astype(jnp.float32)
        emb_sel = emb_sel + sel * emb_ref[k:k + 1, :]
    f_ref[...] = jnp.dot(emb_sel, in_w_ref[...],
                         preferred_element_type=jnp.float32) + in_b_ref[...]
    c_ref[...] = jnp.concatenate(
        [pos_ref[...], jnp.zeros((N_BLK, CW - 3), jnp.float32)], axis=1)


_init_call = pl.pallas_call(
    _init_body,
    grid=(N_NODES // N_BLK,),
    in_specs=[pl.BlockSpec((N_BLK, 1), lambda j: (j, 0)),
              pl.BlockSpec((N_BLK, 3), lambda j: (j, 0)),
              pl.BlockSpec((10, FEAT), lambda j: (0, 0)),
              pl.BlockSpec((FEAT, HID), lambda j: (0, 0)),
              pl.BlockSpec((1, HID), lambda j: (0, 0))],
    out_specs=[pl.BlockSpec((N_BLK, HID), lambda j: (j, 0)),
               pl.BlockSpec((N_BLK, CW), lambda j: (j, 0))],
    out_shape=[jax.ShapeDtypeStruct((N_NODES, HID), jnp.float32),
               jax.ShapeDtypeStruct((N_NODES, CW), jnp.float32)],
)


def _edge_body(xsf_ref, xdf_ref, xcc_ref,
               w1a_ref, w1b_ref, w1c_ref, b1_ref,
               w2_ref, b2_ref, cw1_ref, cb1_ref, cw2_ref, cb2_ref, out_ref):
    fi = xdf_ref[...]
    fj = xsf_ref[...]
    rel = xcc_ref[:, :3] - xcc_ref[:, CW:CW + 3]
    rx = rel[:, 0:1]
    ry = rel[:, 1:2]
    rz = rel[:, 2:3]
    rel_dist = (rx * rx + ry * ry) + rz * rz
    e_in = jnp.concatenate([fi, fj, rel_dist], axis=1)
    w1 = jnp.concatenate([w1a_ref[...], w1b_ref[...], w1c_ref[...]], axis=0)
    pre = jnp.dot(e_in, w1, preferred_element_type=jnp.float32) + b1_ref[...]
    h = _silu(pre)
    m = _silu(jnp.dot(h, w2_ref[...], preferred_element_type=jnp.float32)
              + b2_ref[...])
    t = _silu(jnp.dot(m, cw1_ref[...], preferred_element_type=jnp.float32)
              + cb1_ref[...])
    cw = jnp.dot(t, cw2_ref[...], preferred_element_type=jnp.float32) + cb2_ref[...]
    out_ref[...] = jnp.concatenate(
        [m, cw * rel, jnp.zeros((E_BLK, FEAT - M_DIM - 3), jnp.float32)],
        axis=1)


_EDGE_IN = HID * 2 + 1

_edge_call = pl.pallas_call(
    _edge_body,
    grid=(N_EDGES // E_BLK,),
    in_specs=[pl.BlockSpec((E_BLK, FEAT), lambda j: (j, 0)),
              pl.BlockSpec((E_BLK, FEAT), lambda j: (j, 0)),
              pl.BlockSpec((E_BLK, FEAT), lambda j: (j, 0)),
              pl.BlockSpec((HID, _EDGE_IN * 2), lambda j: (0, 0)),
              pl.BlockSpec((HID, _EDGE_IN * 2), lambda j: (0, 0)),
              pl.BlockSpec((1, _EDGE_IN * 2), lambda j: (0, 0)),
              pl.BlockSpec((1, _EDGE_IN * 2), lambda j: (0, 0)),
              pl.BlockSpec((_EDGE_IN * 2, M_DIM), lambda j: (0, 0)),
              pl.BlockSpec((1, M_DIM), lambda j: (0, 0)),
              pl.BlockSpec((M_DIM, M_DIM * 4), lambda j: (0, 0)),
              pl.BlockSpec((1, M_DIM * 4), lambda j: (0, 0)),
              pl.BlockSpec((M_DIM * 4, 1), lambda j: (0, 0)),
              pl.BlockSpec((1, 1), lambda j: (0, 0))],
    out_specs=pl.BlockSpec((E_BLK, FEAT), lambda j: (j, 0)),
    out_shape=jax.ShapeDtypeStruct((N_EDGES, FEAT), jnp.float32),
)


def _node_body(f_ref, c_ref, a0_ref, a1_ref, nw1_ref, nb1_ref, nw2_ref,
               nb2_ref, fo_ref, co_ref):
    feats = f_ref[...]
    acc = a0_ref[...] + a1_ref[...]
    m_i = acc[:, :M_DIM]
    mhat = acc[:, M_DIM:M_DIM + 3]
    nh = _silu(jnp.dot(jnp.concatenate([feats, m_i], axis=1), nw1_ref[...],
                       preferred_element_type=jnp.float32) + nb1_ref[...])
    fo_ref[...] = feats + jnp.dot(nh, nw2_ref[...],
                                  preferred_element_type=jnp.float32) + nb2_ref[...]
    co_ref[...] = jnp.concatenate(
        [c_ref[:, :3] + mhat, jnp.zeros((N_BLK, CW - 3), jnp.float32)], axis=1)


_node_call = pl.pallas_call(
    _node_body,
    grid=(N_NODES // N_BLK,),
    in_specs=[pl.BlockSpec((N_BLK, HID), lambda j: (j, 0)),
              pl.BlockSpec((N_BLK, CW), lambda j: (j, 0)),
              pl.BlockSpec((N_BLK, EO_W), lambda j: (j, 0)),
              pl.BlockSpec((N_BLK, EO_W),
                           lambda j: (j + N_NODES // N_BLK, 0)),
              pl.BlockSpec((HID + M_DIM, HID * 2), lambda j: (0, 0)),
              pl.BlockSpec((1, HID * 2), lambda j: (0, 0)),
              pl.BlockSpec((HID * 2, HID), lambda j: (0, 0)),
              pl.BlockSpec((1, HID), lambda j: (0, 0))],
    out_specs=[pl.BlockSpec((N_BLK, HID), lambda j: (j, 0)),
               pl.BlockSpec((N_BLK, CW), lambda j: (j, 0))],
    out_shape=[jax.ShapeDtypeStruct((N_NODES, HID), jnp.float32),
               jax.ShapeDtypeStruct((N_NODES, CW), jnp.float32)],
)


def _pool_body(f_ref, b_ref, hw1_ref, hb1_ref, hw2_ref, hb2_ref, res_ref,
               gv_scr):
    j = pl.program_id(0)
    oh = (b_ref[...] == lax.broadcasted_iota(jnp.int32, (N_BLK, N_GRAPHS), 1))
    part = lax.dot_general(oh.astype(jnp.float32), f_ref[...],
                           (((0,), (0,)), ((), ())),
                           preferred_element_type=jnp.float32)

    @pl.when(j == 0)
    def _():
        gv_scr[...] = jnp.zeros((N_GRAPHS, FEAT), jnp.float32)

    gv_scr[...] += part

    @pl.when(j == N_NODES // N_BLK - 1)
    def _():
        hh = _silu(jnp.dot(gv_scr[...], hw1_ref[...],
                           preferred_element_type=jnp.float32) + hb1_ref[...])
        res_ref[...] = (jnp.dot(hh, hw2_ref[...],
                                preferred_element_type=jnp.float32)
                        + hb2_ref[...])


_pool_call = pl.pallas_call(
    _pool_body,
    grid=(N_NODES // N_BLK,),
    in_specs=[pl.BlockSpec((N_BLK, FEAT), lambda j: (j, 0)),
              pl.BlockSpec((N_BLK, 1), lambda j: (j, 0)),
              pl.BlockSpec((HID, HID), lambda j: (0, 0)),
              pl.BlockSpec((1, HID), lambda j: (0, 0)),
              pl.BlockSpec((HID, 1), lambda j: (0, 0)),
              pl.BlockSpec((1, 1), lambda j: (0, 0))],
    out_specs=pl.BlockSpec((N_GRAPHS, 1), lambda j: (0, 0)),
    out_shape=jax.ShapeDtypeStruct((N_GRAPHS, 1), jnp.float32),
    scratch_shapes=[pltpu.VMEM((N_GRAPHS, FEAT), jnp.float32)],
)


def _sc_gather(feats, coors, src1, dst1):
    return _sc_gather_call()(feats, coors, src1, dst1)


def _sc_scatter(eo, dst3, zeros_acc):
    return _sc_scatter_call()(eo, dst3, zeros_acc)


def kernel(z, pos, batch, edge_index, params):
    f32 = jnp.float32
    src1 = edge_index[0].astype(jnp.int32)
    dst1 = edge_index[1].astype(jnp.int32)
    dst3 = dst1.reshape(NW, NCHUNK, CHUNK)
    zeros_acc = jnp.zeros((N_NODES, EO_W), f32)
    z_i = z.astype(jnp.int32).reshape(N_NODES, 1)
    batch_i = batch.astype(jnp.int32).reshape(N_NODES, 1)
    p = params

    feats, coors = _init_call(z_i, pos, p['emb'], p['in_w'],
                              p['in_b'].reshape(1, HID))

    for lp in p['layers']:
        xsf, xdf, xcc = _sc_gather(feats, coors, src1, dst1)
        eo = _edge_call(xsf, xdf, xcc,
                        lp['edge_w1'][:HID],
                        lp['edge_w1'][HID:2 * HID],
                        lp['edge_w1'][2 * HID:].reshape(1, _EDGE_IN * 2),
                        lp['edge_b1'].reshape(1, _EDGE_IN * 2),
                        lp['edge_w2'],
                        lp['edge_b2'].reshape(1, M_DIM),
                        lp['coors_w1'],
                        lp['coors_b1'].reshape(1, M_DIM * 4),
                        lp['coors_w2'],
                        lp['coors_b2'].reshape(1, 1))
        acc = _sc_scatter(eo, dst3, zeros_acc)
        feats, coors = _node_call(feats, coors, acc, acc,
                                  lp['node_w1'], lp['node_b1'].reshape(1, HID * 2),
                                  lp['node_w2'], lp['node_b2'].reshape(1, HID))

    return _pool_call(feats, batch_i, p['head_w1'],
                      p['head_b1'].reshape(1, HID),
                      p['head_w2'], p['head_b2'].reshape(1, 1))


# speedup vs baseline: 2.0224x; 1.0619x over previous
import functools

import jax
import jax.numpy as jnp
from jax import lax
from jax.experimental import pallas as pl
from jax.experimental.pallas import tpu as pltpu
from jax.experimental.pallas import tpu_sc as plsc

N_NODES = 10000
N_EDGES = 320000
FEAT = 128
HID = 128
M_DIM = 16
N_GRAPHS = 64
CW = 8
EO_W = 24

NC = 2
NS = 16
NW = NC * NS
E_PER_W = N_EDGES // NW
CHUNK = 80
NCHUNK = E_PER_W // CHUNK

E_BLK = 2000
N_BLK = 1000

_ROWS_PER_TILE = N_NODES // NS


@functools.cache
def _sc_mesh():
    return plsc.VectorSubcoreMesh(
        core_axis_name="c", subcore_axis_name="s",
        num_cores=NC, num_subcores=NS)


@functools.cache
def _sc_gather_call():
    @functools.partial(
        pl.kernel,
        out_type=[jax.ShapeDtypeStruct((N_EDGES, FEAT), jnp.float32),
                  jax.ShapeDtypeStruct((N_EDGES, FEAT), jnp.float32),
                  jax.ShapeDtypeStruct((N_EDGES, FEAT), jnp.float32)],
        mesh=_sc_mesh(),
        scratch_types=[pltpu.VMEM((E_PER_W,), jnp.int32),
                       pltpu.VMEM((E_PER_W,), jnp.int32),
                       pltpu.VMEM((2, CHUNK, FEAT), jnp.float32),
                       pltpu.VMEM((2, CHUNK, FEAT), jnp.float32),
                       pltpu.VMEM((2, CHUNK, CW), jnp.float32),
                       pltpu.VMEM((2, CHUNK, CW), jnp.float32)]
                      + [pltpu.SemaphoreType.DMA] * 8,
        compiler_params=pltpu.CompilerParams(use_tc_tiling_on_sc=False),
    )
    def gather(feats_hbm, coors_hbm, src_hbm, dst_hbm,
               outs_hbm, outd_hbm, outc_hbm,
               idxs_v, idxd_v, rows_fs, rows_fd, rows_cs, rows_cd,
               gs0, gs1, ws0, ws1, cgs0, cgs1, cws0, cws1):
        wid = lax.axis_index("s") * NC + lax.axis_index("c")
        base = wid * E_PER_W
        pltpu.sync_copy(src_hbm.at[pl.ds(base, E_PER_W)], idxs_v)
        pltpu.sync_copy(dst_hbm.at[pl.ds(base, E_PER_W)], idxd_v)

        def start_gathers(j, b, gsem, cgsem):
            off = j * CHUNK
            pltpu.async_copy(feats_hbm.at[idxs_v.at[pl.ds(off, CHUNK)]],
                             rows_fs.at[b], gsem)
            pltpu.async_copy(feats_hbm.at[idxd_v.at[pl.ds(off, CHUNK)]],
                             rows_fd.at[b], gsem)
            pltpu.async_copy(coors_hbm.at[idxs_v.at[pl.ds(off, CHUNK)]],
                             rows_cs.at[b], cgsem)
            pltpu.async_copy(coors_hbm.at[idxd_v.at[pl.ds(off, CHUNK)]],
                             rows_cd.at[b], cgsem)

        def wait_gathers(b, gsem, cgsem):
            pltpu.make_async_copy(feats_hbm.at[pl.ds(0, CHUNK)],
                                  rows_fs.at[b], gsem).wait()
            pltpu.make_async_copy(feats_hbm.at[pl.ds(0, CHUNK)],
                                  rows_fd.at[b], gsem).wait()
            pltpu.make_async_copy(coors_hbm.at[pl.ds(0, CHUNK)],
                                  rows_cs.at[b], cgsem).wait()
            pltpu.make_async_copy(coors_hbm.at[pl.ds(0, CHUNK)],
                                  rows_cd.at[b], cgsem).wait()

        def start_writes(j, b, wsem, cwsem):
            row = base + j * CHUNK
            pltpu.async_copy(rows_fs.at[b], outs_hbm.at[pl.ds(row, CHUNK)],
                             wsem)
            pltpu.async_copy(rows_fd.at[b], outd_hbm.at[pl.ds(row, CHUNK)],
                             wsem)
            pltpu.async_copy(
                rows_cs.at[b], outc_hbm.at[pl.ds(row, CHUNK), pl.ds(0, CW)],
                cwsem)
            pltpu.async_copy(
                rows_cd.at[b], outc_hbm.at[pl.ds(row, CHUNK), pl.ds(CW, CW)],
                cwsem)

        def wait_writes(b, wsem, cwsem):
            pltpu.make_async_copy(rows_fs.at[b],
                                  outs_hbm.at[pl.ds(base, CHUNK)], wsem).wait()
            pltpu.make_async_copy(rows_fd.at[b],
                                  outd_hbm.at[pl.ds(base, CHUNK)], wsem).wait()
            pltpu.make_async_copy(
                rows_cs.at[b], outc_hbm.at[pl.ds(base, CHUNK), pl.ds(0, CW)],
                cwsem).wait()
            pltpu.make_async_copy(
                rows_cd.at[b], outc_hbm.at[pl.ds(base, CHUNK), pl.ds(CW, CW)],
                cwsem).wait()

        def body(j, carry):
            b = j % 2

            @pl.when(j >= 2)
            def _():
                @pl.when(b == 0)
                def _():
                    wait_writes(0, ws0, cws0)

                @pl.when(b == 1)
                def _():
                    wait_writes(1, ws1, cws1)

            @pl.when(b == 0)
            def _():
                start_gathers(j, 0, gs0, cgs0)

            @pl.when(b == 1)
            def _():
                start_gathers(j, 1, gs1, cgs1)

            @pl.when(j >= 1)
            def _():
                @pl.when(b == 1)
                def _():
                    wait_gathers(0, gs0, cgs0)
                    start_writes(j - 1, 0, ws0, cws0)

                @pl.when(b == 0)
                def _():
                    wait_gathers(1, gs1, cgs1)
                    start_writes(j - 1, 1, ws1, cws1)

            return carry

        lax.fori_loop(0, NCHUNK, body, 0)
        wait_gathers(0, gs0, cgs0)
        start_writes(NCHUNK - 1, 0, ws0, cws0)
        wait_writes(1, ws1, cws1)
        wait_writes(0, ws0, cws0)

    return gather


@functools.cache
def _sc_scatter_call():
    @functools.partial(
        pl.kernel,
        out_type=jax.ShapeDtypeStruct((NC * N_NODES, EO_W), jnp.float32),
        mesh=_sc_mesh(),
        scratch_types=[pltpu.VMEM((NCHUNK, CHUNK), jnp.int32),
                       pltpu.VMEM((CHUNK, EO_W), jnp.float32),
                       pltpu.VMEM_SHARED((N_NODES, EO_W), jnp.float32)],
        compiler_params=pltpu.CompilerParams(use_tc_tiling_on_sc=False),
    )
    def scatter(eo_hbm, dst_hbm, zeros_hbm, acc_hbm, idx_v, rows_v, acc_sp):
        c = lax.axis_index("c")
        s = lax.axis_index("s")
        wid = s * NC + c
        base = wid * E_PER_W
        row0 = s * _ROWS_PER_TILE
        pltpu.sync_copy(zeros_hbm.at[pl.ds(row0, _ROWS_PER_TILE)],
                        acc_sp.at[pl.ds(row0, _ROWS_PER_TILE)])
        pltpu.sync_copy(dst_hbm.at[wid], idx_v)
        plsc.subcore_barrier()

        def body(j, carry):
            pltpu.sync_copy(
                eo_hbm.at[pl.ds(base + j * CHUNK, CHUNK), pl.ds(0, EO_W)],
                rows_v)
            pltpu.sync_copy(rows_v, acc_sp.at[idx_v.at[j]], add=True)
            return carry

        lax.fori_loop(0, NCHUNK, body, 0)
        plsc.subcore_barrier()
        pltpu.sync_copy(acc_sp.at[pl.ds(row0, _ROWS_PER_TILE)],
                        acc_hbm.at[pl.ds(c * N_NODES + row0, _ROWS_PER_TILE)])

    return scatter


def _silu(x):
    return x * jax.nn.sigmoid(x)


def _init_body(z_ref, pos_ref, emb_ref, in_w_ref, in_b_ref, f_ref, c_ref):
    z = z_ref[...]
    emb_sel = jnp.zeros((N_BLK, FEAT), jnp.float32)
    for k in range(10):
        sel = (z == k).astype(jnp.float32)
        emb_sel = emb_sel + sel * emb_ref[k:k + 1, :]
    f_ref[...] = jnp.dot(emb_sel, in_w_ref[...],
                         preferred_element_type=jnp.float32) + in_b_ref[...]
    c_ref[...] = jnp.concatenate(
        [pos_ref[...], jnp.zeros((N_BLK, CW - 3), jnp.float32)], axis=1)


_init_call = pl.pallas_call(
    _init_body,
    grid=(N_NODES // N_BLK,),
    in_specs=[pl.BlockSpec((N_BLK, 1), lambda j: (j, 0)),
              pl.BlockSpec((N_BLK, 3), lambda j: (j, 0)),
              pl.BlockSpec((10, FEAT), lambda j: (0, 0)),
              pl.BlockSpec((FEAT, HID), lambda j: (0, 0)),
              pl.BlockSpec((1, HID), lambda j: (0, 0))],
    out_specs=[pl.BlockSpec((N_BLK, HID), lambda j: (j, 0)),
               pl.BlockSpec((N_BLK, CW), lambda j: (j, 0))],
    out_shape=[jax.ShapeDtypeStruct((N_NODES, HID), jnp.float32),
               jax.ShapeDtypeStruct((N_NODES, CW), jnp.float32)],
)


def _edge_body(xsf_ref, xdf_ref, xcc_ref,
               w1a_ref, w1b_ref, w1c_ref, b1_ref,
               w2_ref, b2_ref, cw1_ref, cb1_ref, cw2_ref, cb2_ref, out_ref):
    fi = xdf_ref[...]
    fj = xsf_ref[...]
    rel = xcc_ref[:, :3] - xcc_ref[:, CW:CW + 3]
    rx = rel[:, 0:1]
    ry = rel[:, 1:2]
    rz = rel[:, 2:3]
    rel_dist = (rx * rx + ry * ry) + rz * rz
    e_in = jnp.concatenate([fi, fj, rel_dist], axis=1)
    w1 = jnp.concatenate([w1a_ref[...], w1b_ref[...], w1c_ref[...]], axis=0)
    pre = jnp.dot(e_in, w1, preferred_element_type=jnp.float32) + b1_ref[...]
    h = _silu(pre)
    m = _silu(jnp.dot(h, w2_ref[...], preferred_element_type=jnp.float32)
              + b2_ref[...])
    t = _silu(jnp.dot(m, cw1_ref[...], preferred_element_type=jnp.float32)
              + cb1_ref[...])
    cw = jnp.dot(t, cw2_ref[...], preferred_element_type=jnp.float32) + cb2_ref[...]
    out_ref[...] = jnp.concatenate(
        [m, cw * rel, jnp.zeros((E_BLK, FEAT - M_DIM - 3), jnp.float32)],
        axis=1)


_EDGE_IN = HID * 2 + 1

_edge_call = pl.pallas_call(
    _edge_body,
    grid=(N_EDGES // E_BLK,),
    in_specs=[pl.BlockSpec((E_BLK, FEAT), lambda j: (j, 0)),
              pl.BlockSpec((E_BLK, FEAT), lambda j: (j, 0)),
              pl.BlockSpec((E_BLK, FEAT), lambda j: (j, 0)),
              pl.BlockSpec((HID, _EDGE_IN * 2), lambda j: (0, 0)),
              pl.BlockSpec((HID, _EDGE_IN * 2), lambda j: (0, 0)),
              pl.BlockSpec((1, _EDGE_IN * 2), lambda j: (0, 0)),
              pl.BlockSpec((1, _EDGE_IN * 2), lambda j: (0, 0)),
              pl.BlockSpec((_EDGE_IN * 2, M_DIM), lambda j: (0, 0)),
              pl.BlockSpec((1, M_DIM), lambda j: (0, 0)),
              pl.BlockSpec((M_DIM, M_DIM * 4), lambda j: (0, 0)),
              pl.BlockSpec((1, M_DIM * 4), lambda j: (0, 0)),
              pl.BlockSpec((M_DIM * 4, 1), lambda j: (0, 0)),
              pl.BlockSpec((1, 1), lambda j: (0, 0))],
    out_specs=pl.BlockSpec((E_BLK, FEAT), lambda j: (j, 0)),
    out_shape=jax.ShapeDtypeStruct((N_EDGES, FEAT), jnp.float32),
)


def _node_body(f_ref, c_ref, a0_ref, a1_ref, nw1_ref, nb1_ref, nw2_ref,
               nb2_ref, fo_ref, co_ref):
    feats = f_ref[...]
    acc = a0_ref[...] + a1_ref[...]
    m_i = acc[:, :M_DIM]
    mhat = acc[:, M_DIM:M_DIM + 3]
    nh = _silu(jnp.dot(jnp.concatenate([feats, m_i], axis=1), nw1_ref[...],
                       preferred_element_type=jnp.float32) + nb1_ref[...])
    fo_ref[...] = feats + jnp.dot(nh, nw2_ref[...],
                                  preferred_element_type=jnp.float32) + nb2_ref[...]
    co_ref[...] = jnp.concatenate(
        [c_ref[:, :3] + mhat, jnp.zeros((N_BLK, CW - 3), jnp.float32)], axis=1)


_node_call = pl.pallas_call(
    _node_body,
    grid=(N_NODES // N_BLK,),
    in_specs=[pl.BlockSpec((N_BLK, HID), lambda j: (j, 0)),
              pl.BlockSpec((N_BLK, CW), lambda j: (j, 0)),
              pl.BlockSpec((N_BLK, EO_W), lambda j: (j, 0)),
              pl.BlockSpec((N_BLK, EO_W),
                           lambda j: (j + N_NODES // N_BLK, 0)),
              pl.BlockSpec((HID + M_DIM, HID * 2), lambda j: (0, 0)),
              pl.BlockSpec((1, HID * 2), lambda j: (0, 0)),
              pl.BlockSpec((HID * 2, HID), lambda j: (0, 0)),
              pl.BlockSpec((1, HID), lambda j: (0, 0))],
    out_specs=[pl.BlockSpec((N_BLK, HID), lambda j: (j, 0)),
               pl.BlockSpec((N_BLK, CW), lambda j: (j, 0))],
    out_shape=[jax.ShapeDtypeStruct((N_NODES, HID), jnp.float32),
               jax.ShapeDtypeStruct((N_NODES, CW), jnp.float32)],
)


def _pool_body(f_ref, b_ref, hw1_ref, hb1_ref, hw2_ref, hb2_ref, res_ref,
               gv_scr):
    j = pl.program_id(0)
    oh = (b_ref[...] == lax.broadcasted_iota(jnp.int32, (N_BLK, N_GRAPHS), 1))
    part = lax.dot_general(oh.astype(jnp.float32), f_ref[...],
                           (((0,), (0,)), ((), ())),
                           preferred_element_type=jnp.float32)

    @pl.when(j == 0)
    def _():
        gv_scr[...] = jnp.zeros((N_GRAPHS, FEAT), jnp.float32)

    gv_scr[...] += part

    @pl.when(j == N_NODES // N_BLK - 1)
    def _():
        hh = _silu(jnp.dot(gv_scr[...], hw1_ref[...],
                           preferred_element_type=jnp.float32) + hb1_ref[...])
        res_ref[...] = (jnp.dot(hh, hw2_ref[...],
                                preferred_element_type=jnp.float32)
                        + hb2_ref[...])


_pool_call = pl.pallas_call(
    _pool_body,
    grid=(N_NODES // N_BLK,),
    in_specs=[pl.BlockSpec((N_BLK, FEAT), lambda j: (j, 0)),
              pl.BlockSpec((N_BLK, 1), lambda j: (j, 0)),
              pl.BlockSpec((HID, HID), lambda j: (0, 0)),
              pl.BlockSpec((1, HID), lambda j: (0, 0)),
              pl.BlockSpec((HID, 1), lambda j: (0, 0)),
              pl.BlockSpec((1, 1), lambda j: (0, 0))],
    out_specs=pl.BlockSpec((N_GRAPHS, 1), lambda j: (0, 0)),
    out_shape=jax.ShapeDtypeStruct((N_GRAPHS, 1), jnp.float32),
    scratch_shapes=[pltpu.VMEM((N_GRAPHS, FEAT), jnp.float32)],
)


def _sc_gather(feats, coors, src1, dst1):
    return _sc_gather_call()(feats, coors, src1, dst1)


def _sc_scatter(eo, dst3, zeros_acc):
    return _sc_scatter_call()(eo, dst3, zeros_acc)


def kernel(z, pos, batch, edge_index, params):
    f32 = jnp.float32
    src1 = edge_index[0].astype(jnp.int32)
    dst1 = edge_index[1].astype(jnp.int32)
    dst3 = dst1.reshape(NW, NCHUNK, CHUNK)
    zeros_acc = jnp.zeros((N_NODES, EO_W), f32)
    z_i = z.astype(jnp.int32).reshape(N_NODES, 1)
    batch_i = batch.astype(jnp.int32).reshape(N_NODES, 1)
    p = params

    feats, coors = _init_call(z_i, pos, p['emb'], p['in_w'],
                              p['in_b'].reshape(1, HID))

    for lp in p['layers']:
        xsf, xdf, xcc = _sc_gather(feats, coors, src1, dst1)
        eo = _edge_call(xsf, xdf, xcc,
                        lp['edge_w1'][:HID],
                        lp['edge_w1'][HID:2 * HID],
                        lp['edge_w1'][2 * HID:].reshape(1, _EDGE_IN * 2),
                        lp['edge_b1'].reshape(1, _EDGE_IN * 2),
                        lp['edge_w2'],
                        lp['edge_b2'].reshape(1, M_DIM),
                        lp['coors_w1'],
                        lp['coors_b1'].reshape(1, M_DIM * 4),
                        lp['coors_w2'],
                        lp['coors_b2'].reshape(1, 1))
        acc = _sc_scatter(eo, dst3, zeros_acc)
        feats, coors = _node_call(feats, coors, acc, acc,
                                  lp['node_w1'], lp['node_b1'].reshape(1, HID * 2),
                                  lp['node_w2'], lp['node_b2'].reshape(1, HID))

    return _pool_call(feats, batch_i, p['head_w1'],
                      p['head_b1'].reshape(1, HID),
                      p['head_w2'], p['head_b2'].reshape(1, 1))
